# Initial kernel scaffold; baseline (speedup 1.0000x reference)
import jax
import jax.numpy as jnp
from jax.experimental import pallas as pl


def kernel(index_sequence, speaker_embedding, tables, is_inference):
    raise NotImplementedError("write your pallas kernel here")


# speedup vs baseline: 9.3707x; 9.3707x over previous
import functools

import jax
import jax.numpy as jnp
from jax import lax
from jax.experimental import pallas as pl
from jax.experimental.pallas import tpu as pltpu
from jax.experimental.pallas import tpu_sc as plsc

NUM_CB = 4
TAB_ROWS = 1030
BT = 8
BATCH = 64
TIME = 2048
OUT_F = NUM_CB * BT
TC = 512
NC = 2
NS = 16
NW = NC * NS


def _body(idx_hbm, tab_hbm, out_hbm, table_v, idx_v, out_v):
  core = lax.axis_index("c")
  sub = lax.axis_index("s")
  wid = sub * NC + core

  pltpu.sync_copy(tab_hbm, table_v)

  iota = lax.iota(jnp.int32, 16)

  for bi in range(BATCH // NW):
    b = wid * (BATCH // NW) + bi
    for ck in range(TIME // TC):
      ts = ck * TC
      for c in range(NUM_CB):
        pltpu.sync_copy(idx_hbm.at[b, c, pl.ds(ts, TC)], idx_v.at[c])

      @pl.loop(0, TC // 16)
      def _(t16):
        t0 = pl.multiple_of(t16 * 16, 16)
        trow32 = (iota + t0) * OUT_F
        for c in range(NUM_CB):
          iv = idx_v[c, pl.ds(t0, 16)]
          base = iv * BT + c * (TAB_ROWS * BT)
          for d in range(BT):
            val = plsc.load_gather(table_v, [base + d])
            plsc.store_scatter(out_v, [trow32 + (c * BT + d)], val)

      pltpu.sync_copy(out_v, out_hbm.at[b, pl.ds(ts * OUT_F, TC * OUT_F)])


@functools.partial(jax.jit, static_argnames=())
def _run(index_sequence, tab_flat):
  mesh = plsc.VectorSubcoreMesh(core_axis_name="c", subcore_axis_name="s")
  fn = pl.kernel(
      _body,
      out_type=jax.ShapeDtypeStruct((BATCH, TIME * OUT_F), jnp.float32),
      mesh=mesh,
      scratch_types=[
          pltpu.VMEM((NUM_CB * TAB_ROWS * BT,), jnp.float32),
          pltpu.VMEM((NUM_CB, TC), jnp.int32),
          pltpu.VMEM((TC * OUT_F,), jnp.float32),
      ],
      compiler_params=pltpu.CompilerParams(needs_layout_passes=False),
  )
  return fn(index_sequence, tab_flat)


def kernel(index_sequence, speaker_embedding, tables, is_inference):
  del speaker_embedding, is_inference
  tab_flat = tables.reshape(-1)
  out = _run(index_sequence, tab_flat)
  return out.reshape(BATCH, TIME, OUT_F)
